# baseline (speedup 1.0000x reference)
import jax
import jax.numpy as jnp
from jax import lax
from jax.experimental import pallas as pl
from jax.experimental.pallas import tpu as pltpu
from jax.experimental.pallas import tpu_sc as plsc

_MASKED_TOKEN = 0
_NUM_CORES = 2
_NUM_SUBCORES = 16
_NUM_WORKERS = _NUM_CORES * _NUM_SUBCORES
_BB = 128
_LANES = 16


def _gather_body(idx_hbm, table_hbm, out_hbm, idxblk_v, col_v, row_v, tp_v, sem):
    t_len, d, nb = out_hbm.shape
    blocks = nb // _BB
    blocks_per_w = blocks // _NUM_WORKERS
    wid = lax.axis_index("s") * _NUM_CORES + lax.axis_index("c")

    lane = lax.iota(jnp.int32, _LANES)
    zeros16 = jnp.zeros((_LANES,), jnp.float32)

    def block_step(bi, carry):
        blk = wid * blocks_per_w + bi
        pltpu.sync_copy(idx_hbm.at[pl.ds(blk * _BB * t_len, _BB * t_len)],
                        idxblk_v)

        def token_step(t, carry2):
            for g in range(_BB // _LANES):
                addr = (g * _LANES + lane) * t_len + t
                col_v[pl.ds(g * _LANES, _LANES)] = plsc.load_gather(
                    idxblk_v, [addr])
            pltpu.async_copy(table_hbm.at[col_v], row_v, sem).wait()

            for g in range(_BB // _LANES):
                vec = col_v[pl.ds(g * _LANES, _LANES)]
                m = vec == _MASKED_TOKEN

                @pl.when(jnp.any(m))
                def _(g=g, m=m):
                    rows = g * _LANES + lane
                    for j in range(d):
                        plsc.store_scatter(
                            row_v,
                            [rows, jnp.full((_LANES,), j, jnp.int32)],
                            zeros16,
                            mask=m,
                        )

            def dim_step(dd, carry3):
                dvec = jnp.full((_LANES,), 0, jnp.int32) + dd
                for g in range(_BB // _LANES):
                    toks = g * _LANES + lane
                    tp_v[dd, pl.ds(g * _LANES, _LANES)] = (
                        plsc.load_gather(row_v, [toks, dvec]))
                return carry3

            lax.fori_loop(0, d, dim_step, 0)

            pltpu.sync_copy(tp_v, out_hbm.at[t, :, pl.ds(blk * _BB, _BB)])
            return carry2

        lax.fori_loop(0, t_len, token_step, 0)
        return carry

    lax.fori_loop(0, blocks_per_w, block_step, 0)


def kernel(indices, embed):
    nb, t = indices.shape
    v, d = embed.shape
    flat_idx = indices.reshape(nb * t).astype(jnp.int32)
    table = jnp.pad(embed, ((0, 0), (0, 128 - d)))
    mesh = plsc.VectorSubcoreMesh(
        core_axis_name="c",
        subcore_axis_name="s",
        num_cores=_NUM_CORES,
        num_subcores=_NUM_SUBCORES,
    )
    run = pl.kernel(
        _gather_body,
        out_type=jax.ShapeDtypeStruct((t, d, nb), jnp.float32),
        mesh=mesh,
        scratch_types=[
            pltpu.VMEM((_BB * t,), jnp.int32),
            pltpu.VMEM((_BB,), jnp.int32),
            pltpu.VMEM((_BB, 128), jnp.float32),
            pltpu.VMEM((d, _BB), jnp.float32),
            pltpu.SemaphoreType.DMA,
        ],
        compiler_params=pltpu.CompilerParams(
            needs_layout_passes=False, use_tc_tiling_on_sc=True
        ),
    )
    out = run(flat_idx, table)
    return jnp.transpose(out, (2, 0, 1))


# speedup vs baseline: 1.0888x; 1.0888x over previous
import jax
import jax.numpy as jnp
from jax import lax
from jax.experimental import pallas as pl
from jax.experimental.pallas import tpu as pltpu
from jax.experimental.pallas import tpu_sc as plsc

_MASKED_TOKEN = 0
_NUM_CORES = 2
_NUM_SUBCORES = 16
_NUM_WORKERS = _NUM_CORES * _NUM_SUBCORES
_BB = 128
_LANES = 16
_NG = _BB // _LANES


def _gather_body(idx_hbm, table_hbm, out_hbm,
                 idxblk_v, col0_v, col1_v, row0_v, row1_v, tp_v,
                 sem0, sem1):
    t_len, d, nb = out_hbm.shape
    blocks = nb // _BB
    blocks_per_w = blocks // _NUM_WORKERS
    wid = lax.axis_index("s") * _NUM_CORES + lax.axis_index("c")

    lane = lax.iota(jnp.int32, _LANES)
    tok_base = [(g * _LANES + lane) * t_len for g in range(_NG)]
    toks = [g * _LANES + lane for g in range(_NG)]
    dvecs = [jnp.full((_LANES,), dd, jnp.int32) for dd in range(d)]

    def fill_col(t, col_v):
        for g in range(_NG):
            col_v[pl.ds(g * _LANES, _LANES)] = plsc.load_gather(
                idxblk_v, [tok_base[g] + t])

    def transpose_store(t, col_v, row_v, blk):
        masks = []
        for g in range(_NG):
            vec = col_v[pl.ds(g * _LANES, _LANES)]
            masks.append(jnp.where(vec == _MASKED_TOKEN, 0.0, 1.0))
        for dd in range(d):
            for g in range(_NG):
                tp_v[dd, pl.ds(g * _LANES, _LANES)] = (
                    plsc.load_gather(row_v, [toks[g], dvecs[dd]]) * masks[g])
        pltpu.sync_copy(tp_v, out_hbm.at[t, :, pl.ds(blk * _BB, _BB)])

    def block_step(bi, carry):
        blk = wid * blocks_per_w + bi
        pltpu.sync_copy(idx_hbm.at[pl.ds(blk * _BB * t_len, _BB * t_len)],
                        idxblk_v)

        fill_col(0, col0_v)
        pltpu.async_copy(table_hbm.at[col0_v], row0_v, sem0)

        def token_pair(i, carry2):
            t0 = 2 * i
            fill_col(t0 + 1, col1_v)
            pltpu.async_copy(table_hbm.at[col1_v], row1_v, sem1)
            pltpu.make_async_copy(table_hbm.at[col0_v], row0_v, sem0).wait()
            transpose_store(t0, col0_v, row0_v, blk)
            t2 = jnp.minimum(t0 + 2, t_len - 1)
            fill_col(t2, col0_v)
            pltpu.async_copy(table_hbm.at[col0_v], row0_v, sem0)
            pltpu.make_async_copy(table_hbm.at[col1_v], row1_v, sem1).wait()
            transpose_store(t0 + 1, col1_v, row1_v, blk)
            return carry2

        lax.fori_loop(0, t_len // 2, token_pair, 0)
        pltpu.make_async_copy(table_hbm.at[col0_v], row0_v, sem0).wait()
        return carry

    lax.fori_loop(0, blocks_per_w, block_step, 0)


def kernel(indices, embed):
    nb, t = indices.shape
    v, d = embed.shape
    flat_idx = indices.reshape(nb * t).astype(jnp.int32)
    table = jnp.pad(embed, ((0, 0), (0, 128 - d)))
    mesh = plsc.VectorSubcoreMesh(
        core_axis_name="c",
        subcore_axis_name="s",
        num_cores=_NUM_CORES,
        num_subcores=_NUM_SUBCORES,
    )
    run = pl.kernel(
        _gather_body,
        out_type=jax.ShapeDtypeStruct((t, d, nb), jnp.float32),
        mesh=mesh,
        scratch_types=[
            pltpu.VMEM((_BB * t,), jnp.int32),
            pltpu.VMEM((_BB,), jnp.int32),
            pltpu.VMEM((_BB,), jnp.int32),
            pltpu.VMEM((_BB, 128), jnp.float32),
            pltpu.VMEM((_BB, 128), jnp.float32),
            pltpu.VMEM((d, _BB), jnp.float32),
            pltpu.SemaphoreType.DMA,
            pltpu.SemaphoreType.DMA,
        ],
        compiler_params=pltpu.CompilerParams(
            needs_layout_passes=False, use_tc_tiling_on_sc=True
        ),
    )
    out = run(flat_idx, table)
    return jnp.transpose(out, (2, 0, 1))


# speedup vs baseline: 1.1294x; 1.0374x over previous
import jax
import jax.numpy as jnp
from jax import lax
from jax.experimental import pallas as pl
from jax.experimental.pallas import tpu as pltpu
from jax.experimental.pallas import tpu_sc as plsc

_MASKED_TOKEN = 0
_NUM_CORES = 2
_NUM_SUBCORES = 16
_NUM_WORKERS = _NUM_CORES * _NUM_SUBCORES
_BB = 128
_LANES = 16
_NG = _BB // _LANES


def _gather_body(idx_hbm, table_hbm, out_hbm,
                 idxblk_v, col0_v, col1_v, row0_v, row1_v, tp_v,
                 sem0, sem1):
    t_len, d, nb = out_hbm.shape
    blocks = nb // _BB
    blocks_per_w = blocks // _NUM_WORKERS
    wid = lax.axis_index("s") * _NUM_CORES + lax.axis_index("c")

    lane = lax.iota(jnp.int32, _LANES)
    tok_base = [(g * _LANES + lane) * t_len for g in range(_NG)]
    toks = [g * _LANES + lane for g in range(_NG)]
    zero16 = jnp.zeros((_LANES,), jnp.int32)

    def fill_col(t, col_v):
        for g in range(_NG):
            col_v[pl.ds(g * _LANES, _LANES)] = plsc.load_gather(
                idxblk_v, [tok_base[g] + t])

    def transpose_store(t, col_v, row_v, blk):
        masks = []
        for g in range(_NG):
            vec = col_v[pl.ds(g * _LANES, _LANES)]
            masks.append(jnp.where(vec == _MASKED_TOKEN, 0.0, 1.0))
        def dim4_step(q, carry3):
            d0 = q * 4
            dvec0 = zero16 + d0
            for k in range(4):
                dvec = dvec0 + k
                for g in range(_NG):
                    tp_v[d0 + k, pl.ds(g * _LANES, _LANES)] = (
                        plsc.load_gather(row_v, [toks[g], dvec]) * masks[g])
            return carry3

        lax.fori_loop(0, d // 4, dim4_step, 0)
        pltpu.sync_copy(tp_v, out_hbm.at[t, :, pl.ds(blk * _BB, _BB)])

    def block_step(bi, carry):
        blk = wid * blocks_per_w + bi
        pltpu.sync_copy(idx_hbm.at[pl.ds(blk * _BB * t_len, _BB * t_len)],
                        idxblk_v)

        fill_col(0, col0_v)
        pltpu.async_copy(table_hbm.at[col0_v], row0_v, sem0)

        def token_pair(i, carry2):
            t0 = 2 * i
            fill_col(t0 + 1, col1_v)
            pltpu.async_copy(table_hbm.at[col1_v], row1_v, sem1)
            pltpu.make_async_copy(table_hbm.at[col0_v], row0_v, sem0).wait()
            transpose_store(t0, col0_v, row0_v, blk)
            t2 = jnp.minimum(t0 + 2, t_len - 1)
            fill_col(t2, col0_v)
            pltpu.async_copy(table_hbm.at[col0_v], row0_v, sem0)
            pltpu.make_async_copy(table_hbm.at[col1_v], row1_v, sem1).wait()
            transpose_store(t0 + 1, col1_v, row1_v, blk)
            return carry2

        lax.fori_loop(0, t_len // 2, token_pair, 0)
        pltpu.make_async_copy(table_hbm.at[col0_v], row0_v, sem0).wait()
        return carry

    lax.fori_loop(0, blocks_per_w, block_step, 0)


def kernel(indices, embed):
    nb, t = indices.shape
    v, d = embed.shape
    flat_idx = indices.reshape(nb * t).astype(jnp.int32)
    table = jnp.pad(embed, ((0, 0), (0, 128 - d)))
    mesh = plsc.VectorSubcoreMesh(
        core_axis_name="c",
        subcore_axis_name="s",
        num_cores=_NUM_CORES,
        num_subcores=_NUM_SUBCORES,
    )
    run = pl.kernel(
        _gather_body,
        out_type=jax.ShapeDtypeStruct((t, d, nb), jnp.float32),
        mesh=mesh,
        scratch_types=[
            pltpu.VMEM((_BB * t,), jnp.int32),
            pltpu.VMEM((_BB,), jnp.int32),
            pltpu.VMEM((_BB,), jnp.int32),
            pltpu.VMEM((_BB, 128), jnp.float32),
            pltpu.VMEM((_BB, 128), jnp.float32),
            pltpu.VMEM((d, _BB), jnp.float32),
            pltpu.SemaphoreType.DMA,
            pltpu.SemaphoreType.DMA,
        ],
        compiler_params=pltpu.CompilerParams(
            needs_layout_passes=False, use_tc_tiling_on_sc=True
        ),
    )
    out = run(flat_idx, table)
    return jnp.transpose(out, (2, 0, 1))


# speedup vs baseline: 1.1805x; 1.0452x over previous
import jax
import jax.numpy as jnp
from jax import lax
from jax.experimental import pallas as pl
from jax.experimental.pallas import tpu as pltpu
from jax.experimental.pallas import tpu_sc as plsc

_MASKED_TOKEN = 0
_NUM_CORES = 2
_NUM_SUBCORES = 16
_NUM_WORKERS = _NUM_CORES * _NUM_SUBCORES
_BB = 128
_LANES = 16
_NG = _BB // _LANES
_DEPTH = 4


def _gather_body(idx_hbm, table_hbm, out_hbm,
                 idxblk_v, cols_v, rows_v, tp0_v, tp1_v, semg, semw):
    t_len, d, nb = out_hbm.shape
    blocks = nb // _BB
    blocks_per_w = blocks // _NUM_WORKERS
    wid = lax.axis_index("s") * _NUM_CORES + lax.axis_index("c")

    lane = lax.iota(jnp.int32, _LANES)
    tok_base = [(g * _LANES + lane) * t_len for g in range(_NG)]
    toks = [g * _LANES + lane for g in range(_NG)]
    zero16 = jnp.zeros((_LANES,), jnp.int32)
    tps = [tp0_v, tp1_v]

    def fill_col(t, col_v):
        for g in range(_NG):
            col_v[pl.ds(g * _LANES, _LANES)] = plsc.load_gather(
                idxblk_v, [tok_base[g] + t])

    def transpose(col_v, row_v, tp_v):
        masks = []
        for g in range(_NG):
            vec = col_v[pl.ds(g * _LANES, _LANES)]
            masks.append(jnp.where(vec == _MASKED_TOKEN, 0.0, 1.0))

        def dim4_step(q, carry3):
            d0 = q * 4
            dvec0 = zero16 + d0
            for k in range(4):
                dvec = dvec0 + k
                for g in range(_NG):
                    tp_v[d0 + k, pl.ds(g * _LANES, _LANES)] = (
                        plsc.load_gather(row_v, [toks[g], dvec]) * masks[g])
            return carry3

        lax.fori_loop(0, d // 4, dim4_step, 0)

    def block_step(bi, carry):
        blk = wid * blocks_per_w + bi
        pltpu.sync_copy(idx_hbm.at[pl.ds(blk * _BB * t_len, _BB * t_len)],
                        idxblk_v)

        for r in range(_DEPTH):
            fill_col(r, cols_v.at[r])
            pltpu.async_copy(table_hbm.at[cols_v.at[r]], rows_v.at[r], semg)

        def token_quad(i, carry2):
            for r in range(_DEPTH):
                t = _DEPTH * i + r
                p = r % 2
                pltpu.make_async_copy(
                    table_hbm.at[cols_v.at[r]], rows_v.at[r], semg).wait()

                @pl.when(_DEPTH * i + r >= 2)
                def _(p=p, t=t, blk=blk):
                    pltpu.make_async_copy(
                        tps[p],
                        out_hbm.at[jnp.maximum(t - 2, 0), :,
                                   pl.ds(blk * _BB, _BB)],
                        semw).wait()

                transpose(cols_v.at[r], rows_v.at[r], tps[p])
                pltpu.async_copy(
                    tps[p], out_hbm.at[t, :, pl.ds(blk * _BB, _BB)], semw)

                tn = jnp.minimum(t + _DEPTH, t_len - 1)
                fill_col(tn, cols_v.at[r])
                pltpu.async_copy(
                    table_hbm.at[cols_v.at[r]], rows_v.at[r], semg)
            return carry2

        lax.fori_loop(0, t_len // _DEPTH, token_quad, 0)

        for r in range(_DEPTH):
            pltpu.make_async_copy(
                table_hbm.at[cols_v.at[r]], rows_v.at[r], semg).wait()
        for p in range(2):
            pltpu.make_async_copy(
                tps[p], out_hbm.at[0, :, pl.ds(blk * _BB, _BB)], semw).wait()
        return carry

    lax.fori_loop(0, blocks_per_w, block_step, 0)


def kernel(indices, embed):
    nb, t = indices.shape
    v, d = embed.shape
    flat_idx = indices.reshape(nb * t).astype(jnp.int32)
    table = jnp.pad(embed, ((0, 0), (0, 128 - d)))
    mesh = plsc.VectorSubcoreMesh(
        core_axis_name="c",
        subcore_axis_name="s",
        num_cores=_NUM_CORES,
        num_subcores=_NUM_SUBCORES,
    )
    run = pl.kernel(
        _gather_body,
        out_type=jax.ShapeDtypeStruct((t, d, nb), jnp.float32),
        mesh=mesh,
        scratch_types=[
            pltpu.VMEM((_BB * t,), jnp.int32),
            pltpu.VMEM((_DEPTH, _BB), jnp.int32),
            pltpu.VMEM((_DEPTH, _BB, 128), jnp.float32),
            pltpu.VMEM((d, _BB), jnp.float32),
            pltpu.VMEM((d, _BB), jnp.float32),
            pltpu.SemaphoreType.DMA,
            pltpu.SemaphoreType.DMA,
        ],
        compiler_params=pltpu.CompilerParams(
            needs_layout_passes=False, use_tc_tiling_on_sc=True
        ),
    )
    out = run(flat_idx, table)
    return jnp.transpose(out, (2, 0, 1))


# speedup vs baseline: 2.1210x; 1.7968x over previous
import jax
import jax.numpy as jnp
from jax import lax
from jax.experimental import pallas as pl
from jax.experimental.pallas import tpu as pltpu
from jax.experimental.pallas import tpu_sc as plsc

_MASKED_TOKEN = 0
_NUM_CORES = 2
_NUM_SUBCORES = 16
_NUM_WORKERS = _NUM_CORES * _NUM_SUBCORES
_BB = 128
_LANES = 16
_NG = _BB // _LANES
_DEPTH = 4


def _gather_body(idx_hbm, table_hbm, out_hbm,
                 idxblk_v, cols_v, rows_v, tp0_v, tp1_v, semg, semw):
    t_len, d, nb = out_hbm.shape
    blocks = nb // _BB
    blocks_per_w = blocks // _NUM_WORKERS
    wid = lax.axis_index("s") * _NUM_CORES + lax.axis_index("c")

    lane = lax.iota(jnp.int32, _LANES)
    tok_base = [(g * _LANES + lane) * t_len for g in range(_NG)]
    toks = [g * _LANES + lane for g in range(_NG)]
    zero16 = jnp.zeros((_LANES,), jnp.int32)
    tps = [tp0_v, tp1_v]

    def fill_col(t, col_v):
        for g in range(_NG):
            col_v[pl.ds(g * _LANES, _LANES)] = plsc.load_gather(
                idxblk_v, [tok_base[g] + t])

    def transpose(col_v, row_v, tp_v):
        masks = []
        for g in range(_NG):
            vec = col_v[pl.ds(g * _LANES, _LANES)]
            masks.append(jnp.where(vec == _MASKED_TOKEN, 0.0, 1.0))

        @plsc.parallel_loop(0, d // 4, unroll=2)
        def _(q):
            d0 = q * 4
            dvec0 = zero16 + d0
            for k in range(4):
                dvec = dvec0 + k
                for g in range(_NG):
                    tp_v[d0 + k, pl.ds(g * _LANES, _LANES)] = (
                        plsc.load_gather(row_v, [toks[g], dvec]) * masks[g])

    def block_step(bi, carry):
        blk = wid * blocks_per_w + bi
        pltpu.sync_copy(idx_hbm.at[pl.ds(blk * _BB * t_len, _BB * t_len)],
                        idxblk_v)

        for r in range(_DEPTH):
            fill_col(r, cols_v.at[r])
            pltpu.async_copy(table_hbm.at[cols_v.at[r]], rows_v.at[r], semg)

        def token_quad(i, carry2):
            for r in range(_DEPTH):
                t = _DEPTH * i + r
                p = r % 2
                pltpu.make_async_copy(
                    table_hbm.at[cols_v.at[r]], rows_v.at[r], semg).wait()

                @pl.when(_DEPTH * i + r >= 2)
                def _(p=p, t=t, blk=blk):
                    pltpu.make_async_copy(
                        tps[p],
                        out_hbm.at[jnp.maximum(t - 2, 0), :,
                                   pl.ds(blk * _BB, _BB)],
                        semw).wait()

                transpose(cols_v.at[r], rows_v.at[r], tps[p])
                pltpu.async_copy(
                    tps[p], out_hbm.at[t, :, pl.ds(blk * _BB, _BB)], semw)

                tn = jnp.minimum(t + _DEPTH, t_len - 1)
                fill_col(tn, cols_v.at[r])
                pltpu.async_copy(
                    table_hbm.at[cols_v.at[r]], rows_v.at[r], semg)
            return carry2

        lax.fori_loop(0, t_len // _DEPTH, token_quad, 0)

        for r in range(_DEPTH):
            pltpu.make_async_copy(
                table_hbm.at[cols_v.at[r]], rows_v.at[r], semg).wait()
        for p in range(2):
            pltpu.make_async_copy(
                tps[p], out_hbm.at[0, :, pl.ds(blk * _BB, _BB)], semw).wait()
        return carry

    lax.fori_loop(0, blocks_per_w, block_step, 0)


def kernel(indices, embed):
    nb, t = indices.shape
    v, d = embed.shape
    flat_idx = indices.reshape(nb * t).astype(jnp.int32)
    table = jnp.pad(embed, ((0, 0), (0, 128 - d)))
    mesh = plsc.VectorSubcoreMesh(
        core_axis_name="c",
        subcore_axis_name="s",
        num_cores=_NUM_CORES,
        num_subcores=_NUM_SUBCORES,
    )
    run = pl.kernel(
        _gather_body,
        out_type=jax.ShapeDtypeStruct((t, d, nb), jnp.float32),
        mesh=mesh,
        scratch_types=[
            pltpu.VMEM((_BB * t,), jnp.int32),
            pltpu.VMEM((_DEPTH, _BB), jnp.int32),
            pltpu.VMEM((_DEPTH, _BB, 128), jnp.float32),
            pltpu.VMEM((d, _BB), jnp.float32),
            pltpu.VMEM((d, _BB), jnp.float32),
            pltpu.SemaphoreType.DMA,
            pltpu.SemaphoreType.DMA,
        ],
        compiler_params=pltpu.CompilerParams(
            needs_layout_passes=False, use_tc_tiling_on_sc=True
        ),
    )
    out = run(flat_idx, table)
    return jnp.transpose(out, (2, 0, 1))


# speedup vs baseline: 2.1603x; 1.0185x over previous
import jax
import jax.numpy as jnp
from jax import lax
from jax.experimental import pallas as pl
from jax.experimental.pallas import tpu as pltpu
from jax.experimental.pallas import tpu_sc as plsc

_MASKED_TOKEN = 0
_NUM_CORES = 2
_NUM_SUBCORES = 16
_NUM_WORKERS = _NUM_CORES * _NUM_SUBCORES
_BB = 128
_LANES = 16
_NG = _BB // _LANES
_DEPTH = 4


def _gather_body(idx_hbm, table_hbm, out_hbm,
                 idxblk_v, cols_v, rows_v, tp0_v, tp1_v, semg, semw):
    t_len, d, nb = out_hbm.shape
    blocks = nb // _BB
    blocks_per_w = blocks // _NUM_WORKERS
    wid = lax.axis_index("s") * _NUM_CORES + lax.axis_index("c")

    lane = lax.iota(jnp.int32, _LANES)
    tok_base = [(g * _LANES + lane) * t_len for g in range(_NG)]
    toks = [g * _LANES + lane for g in range(_NG)]
    zero16 = jnp.zeros((_LANES,), jnp.int32)
    tps = [tp0_v, tp1_v]

    def fill_col(t, col_v):
        for g in range(_NG):
            col_v[pl.ds(g * _LANES, _LANES)] = plsc.load_gather(
                idxblk_v, [tok_base[g] + t])

    def transpose(col_v, row_v, tp_v):
        masks = []
        for g in range(_NG):
            vec = col_v[pl.ds(g * _LANES, _LANES)]
            masks.append(jnp.where(vec == _MASKED_TOKEN, 0.0, 1.0))

        @plsc.parallel_loop(0, d // 4, unroll=4)
        def _(q):
            d0 = q * 4
            dvec0 = zero16 + d0
            for k in range(4):
                dvec = dvec0 + k
                for g in range(_NG):
                    tp_v[d0 + k, pl.ds(g * _LANES, _LANES)] = (
                        plsc.load_gather(row_v, [toks[g], dvec]) * masks[g])

    def block_step(bi, carry):
        blk = wid * blocks_per_w + bi
        pltpu.sync_copy(idx_hbm.at[pl.ds(blk * _BB * t_len, _BB * t_len)],
                        idxblk_v)

        for r in range(_DEPTH):
            fill_col(r, cols_v.at[r])
            pltpu.async_copy(table_hbm.at[cols_v.at[r]], rows_v.at[r], semg)

        def token_quad(i, carry2):
            for r in range(_DEPTH):
                t = _DEPTH * i + r
                p = r % 2
                pltpu.make_async_copy(
                    table_hbm.at[cols_v.at[r]], rows_v.at[r], semg).wait()

                @pl.when(_DEPTH * i + r >= 2)
                def _(p=p, t=t, blk=blk):
                    pltpu.make_async_copy(
                        tps[p],
                        out_hbm.at[jnp.maximum(t - 2, 0), :,
                                   pl.ds(blk * _BB, _BB)],
                        semw).wait()

                transpose(cols_v.at[r], rows_v.at[r], tps[p])
                pltpu.async_copy(
                    tps[p], out_hbm.at[t, :, pl.ds(blk * _BB, _BB)], semw)

                tn = jnp.minimum(t + _DEPTH, t_len - 1)
                fill_col(tn, cols_v.at[r])
                pltpu.async_copy(
                    table_hbm.at[cols_v.at[r]], rows_v.at[r], semg)
            return carry2

        lax.fori_loop(0, t_len // _DEPTH, token_quad, 0)

        for r in range(_DEPTH):
            pltpu.make_async_copy(
                table_hbm.at[cols_v.at[r]], rows_v.at[r], semg).wait()
        for p in range(2):
            pltpu.make_async_copy(
                tps[p], out_hbm.at[0, :, pl.ds(blk * _BB, _BB)], semw).wait()
        return carry

    lax.fori_loop(0, blocks_per_w, block_step, 0)


def kernel(indices, embed):
    nb, t = indices.shape
    v, d = embed.shape
    flat_idx = indices.reshape(nb * t).astype(jnp.int32)
    table = jnp.pad(embed, ((0, 0), (0, 128 - d)))
    mesh = plsc.VectorSubcoreMesh(
        core_axis_name="c",
        subcore_axis_name="s",
        num_cores=_NUM_CORES,
        num_subcores=_NUM_SUBCORES,
    )
    run = pl.kernel(
        _gather_body,
        out_type=jax.ShapeDtypeStruct((t, d, nb), jnp.float32),
        mesh=mesh,
        scratch_types=[
            pltpu.VMEM((_BB * t,), jnp.int32),
            pltpu.VMEM((_DEPTH, _BB), jnp.int32),
            pltpu.VMEM((_DEPTH, _BB, 128), jnp.float32),
            pltpu.VMEM((d, _BB), jnp.float32),
            pltpu.VMEM((d, _BB), jnp.float32),
            pltpu.SemaphoreType.DMA,
            pltpu.SemaphoreType.DMA,
        ],
        compiler_params=pltpu.CompilerParams(
            needs_layout_passes=False, use_tc_tiling_on_sc=True
        ),
    )
    out = run(flat_idx, table)
    return jnp.transpose(out, (2, 0, 1))
